# Initial kernel scaffold; baseline (speedup 1.0000x reference)
import jax
import jax.numpy as jnp
from jax.experimental import pallas as pl


def kernel(pcl_noisy, pcl_clean, feat_W1, feat_b1, feat_W2, feat_b2, score_Win, score_bin, score_Wb, score_bb, score_Wout, score_bout):
    raise NotImplementedError("write your pallas kernel here")


# speedup vs baseline: 1.6824x; 1.6824x over previous
import jax
import jax.numpy as jnp
from jax import lax
from jax.experimental import pallas as pl
from jax.experimental.pallas import tpu as pltpu
from jax.experimental.pallas import tpu_sc as plsc

_N_NOISY = 50000
_N_CLEAN = 52000
_NQ = 128
_K = 32
_C = 4
_F = 128
_HID = 128
_NBLK = 4
_SIGMA = 0.01

_N1_PAD = 51200
_N1_CHUNK = 12800
_N1_NCHUNK = 4
_N2_PAD = 52224
_N2_CHUNK = 6528
_N2_NCHUNK = 8
_QTILE = 256
_BIGF = 3.0e38
_BIGI = 2 ** 30


def _extract_min(d2, iota):
    m = jnp.min(d2, axis=1, keepdims=True)
    pos = jnp.min(jnp.where(d2 == m, iota, _BIGI), axis=1, keepdims=True)
    d2n = jnp.where(iota == pos, _BIGF, d2)
    return m, pos, d2n


def _knn1_body(q_ref, xyc_ref, idx_ref):
    q = q_ref[...]
    qsq = jnp.sum(q * q, axis=1, keepdims=True)
    nslot = _N1_NCHUNK * _K
    slot_iota = lax.broadcasted_iota(jnp.int32, (1, nslot), 1)

    def chunk_body(c, carry):
        cv, ci = carry
        ch = xyc_ref[c]
        psq = jnp.sum(ch * ch, axis=0, keepdims=True)
        d2 = qsq - 2.0 * jnp.dot(q, ch, preferred_element_type=jnp.float32) + psq
        iota = lax.broadcasted_iota(jnp.int32, d2.shape, 1)

        def k_body(k, kc):
            d2k, cv, ci = kc
            m, pos, d2k = _extract_min(d2k, iota)
            sel = slot_iota == (c * _K + k)
            cv = jnp.where(sel, m, cv)
            ci = jnp.where(sel, pos + c * _N1_CHUNK, ci)
            return d2k, cv, ci

        _, cv, ci = lax.fori_loop(0, _K, k_body, (d2, cv, ci))
        return cv, ci

    cv0 = jnp.full((_NQ, nslot), _BIGF, jnp.float32)
    ci0 = jnp.zeros((_NQ, nslot), jnp.int32)
    cv, ci = lax.fori_loop(0, _N1_NCHUNK, chunk_body, (cv0, ci0))

    iota2 = lax.broadcasted_iota(jnp.int32, cv.shape, 1)
    kiota = lax.broadcasted_iota(jnp.int32, (1, _K), 1)

    def out_body(k, oc):
        cv, out = oc
        m = jnp.min(cv, axis=1, keepdims=True)
        pos = jnp.min(jnp.where(cv == m, iota2, _BIGI), axis=1, keepdims=True)
        sel = iota2 == pos
        gi = jnp.sum(jnp.where(sel, ci, 0), axis=1, keepdims=True)
        out = jnp.where(kiota == k, gi, out)
        cv = jnp.where(sel, _BIGF, cv)
        return cv, out

    _, out = lax.fori_loop(0, _K, out_body,
                           (cv, jnp.zeros((_NQ, _K), jnp.int32)))
    idx_ref[...] = out


def _knn1(qxy8, xyc):
    return pl.pallas_call(
        _knn1_body,
        out_shape=jax.ShapeDtypeStruct((_NQ, _K), jnp.int32),
    )(qxy8, xyc)


def _sc_gather_body(table_hbm, idx_hbm, out_hbm, idx_v, rows_v, sem):
    wid = lax.axis_index("s") * 2 + lax.axis_index("c")
    base = wid * 128
    pltpu.sync_copy(idx_hbm.at[pl.ds(base, 128)], idx_v)
    pltpu.async_copy(table_hbm.at[idx_v], rows_v, sem).wait()
    pltpu.sync_copy(rows_v, out_hbm.at[pl.ds(base, 128)])


def _sc_gather(table128, idxf):
    mesh = plsc.VectorSubcoreMesh(core_axis_name="c", subcore_axis_name="s")
    k = pl.kernel(
        _sc_gather_body,
        out_type=jax.ShapeDtypeStruct((_NQ * _K, 128), jnp.float32),
        mesh=mesh,
        scratch_types=[
            pltpu.VMEM((128,), jnp.int32),
            pltpu.VMEM((128, 128), jnp.float32),
            pltpu.SemaphoreType.DMA,
        ],
    )
    return k(table128, idxf)


def _knn2_body(fq_ref, cc_ref, mz_ref):
    q = fq_ref[...]
    qsq = jnp.sum(q * q, axis=1, keepdims=True)
    nslot = _N2_NCHUNK * _C
    slot_iota = lax.broadcasted_iota(jnp.int32, (1, nslot), 1)

    def chunk_body(c, carry):
        cv, cz = carry
        ch = cc_ref[c]
        xy = ch[0:2]
        psq = jnp.sum(xy * xy, axis=0, keepdims=True)
        zrow = ch[2:3]
        d2 = qsq - 2.0 * jnp.dot(q, ch, preferred_element_type=jnp.float32) + psq
        iota = lax.broadcasted_iota(jnp.int32, d2.shape, 1)

        def k_body(k, kc):
            d2k, cv, cz = kc
            m, pos, d2k = _extract_min(d2k, iota)
            z = jnp.sum(jnp.where(iota == pos, zrow, 0.0), axis=1, keepdims=True)
            sel = slot_iota == (c * _C + k)
            cv = jnp.where(sel, m, cv)
            cz = jnp.where(sel, z, cz)
            return d2k, cv, cz

        _, cv, cz = lax.fori_loop(0, _C, k_body, (d2, cv, cz))
        return cv, cz

    cv0 = jnp.full((_QTILE, nslot), _BIGF, jnp.float32)
    cz0 = jnp.zeros((_QTILE, nslot), jnp.float32)
    cv, cz = lax.fori_loop(0, _N2_NCHUNK, chunk_body, (cv0, cz0))

    iota2 = lax.broadcasted_iota(jnp.int32, cv.shape, 1)

    def out_body(k, oc):
        cv, acc = oc
        m = jnp.min(cv, axis=1, keepdims=True)
        pos = jnp.min(jnp.where(cv == m, iota2, _BIGI), axis=1, keepdims=True)
        sel = iota2 == pos
        acc = acc + jnp.sum(jnp.where(sel, cz, 0.0), axis=1, keepdims=True)
        cv = jnp.where(sel, _BIGF, cv)
        return cv, acc

    _, acc = lax.fori_loop(0, _C, out_body,
                           (cv, jnp.zeros((_QTILE, 1), jnp.float32)))
    mz_ref[...] = acc * (1.0 / _C)


def _knn2(fq, cc):
    n = _NQ * _K
    return pl.pallas_call(
        _knn2_body,
        grid=(n // _QTILE,),
        in_specs=[
            pl.BlockSpec((_QTILE, 8), lambda i: (i, 0)),
            pl.BlockSpec((_N2_NCHUNK, 8, _N2_CHUNK), lambda i: (0, 0, 0)),
        ],
        out_specs=pl.BlockSpec((_QTILE, 1), lambda i: (i, 0)),
        out_shape=jax.ShapeDtypeStruct((n, 1), jnp.float32),
    )(fq, cc)


def _mlp_body(q_ref, fr_ref, mz_ref, w1_ref, b1_ref, w2_ref, b2_ref,
              wa_ref, wbf_ref, bin_ref, wbk_ref, bbk_ref, wo_ref, bo_ref,
              out_ref):
    q = q_ref[...]
    h = jnp.maximum(jnp.dot(q, w1_ref[...], preferred_element_type=jnp.float32)
                    + b1_ref[...], 0.0)
    feat = jnp.dot(h, w2_ref[...], preferred_element_type=jnp.float32) + b2_ref[...]
    fr = fr_ref[...]

    def rep(a):
        return jnp.reshape(
            jnp.broadcast_to(a[:, None, :], (_NQ, _K, a.shape[1])),
            (_NQ * _K, a.shape[1]))

    cfeat = rep(feat)
    x = fr - rep(q)
    hs = jnp.maximum(
        jnp.dot(x, wa_ref[...], preferred_element_type=jnp.float32)
        + jnp.dot(cfeat, wbf_ref[...], preferred_element_type=jnp.float32)
        + bin_ref[...], 0.0)
    for i in range(_NBLK):
        hs = jnp.maximum(
            jnp.dot(hs, wbk_ref[i], preferred_element_type=jnp.float32)
            + bbk_ref[i], 0.0) + hs
    gp = jnp.dot(hs, wo_ref[...], preferred_element_type=jnp.float32)
    colmask = lax.broadcasted_iota(jnp.int32, gp.shape, 1) == 0
    z = fr[:, 2:3]
    gt = mz_ref[...] - z
    diff = jnp.where(colmask, gt - (gp + bo_ref[...]), 0.0)
    total = jnp.sum(diff * diff)
    out_ref[...] = jnp.reshape(0.5 * total * (1.0 / _SIGMA) / (_NQ * _K), (1, 1))


def _mlp(q16, frames16, mz, w1p, b1, w2, b2, wina, winb, binp, wbk, bbk, wop, bop):
    return pl.pallas_call(
        _mlp_body,
        out_shape=jax.ShapeDtypeStruct((1, 1), jnp.float32),
    )(q16, frames16, mz, w1p, b1, w2, b2, wina, winb, binp, wbk, bbk, wop, bop)


def kernel(pcl_noisy, pcl_clean, feat_W1, feat_b1, feat_W2, feat_b2,
           score_Win, score_bin, score_Wb, score_bb, score_Wout, score_bout):
    f32 = jnp.float32
    pn = pcl_noisy[0]
    pc = pcl_clean[0]

    table128 = jnp.pad(pn, ((0, 0), (0, 125)))
    q16 = jnp.pad(pn[:_NQ], ((0, 0), (0, 13)))
    qxy8 = jnp.concatenate(
        [pn[:_NQ, :2], jnp.zeros((_NQ, 6), f32)], axis=1)

    xyT = jnp.concatenate([pn[:, :2].T, jnp.zeros((6, _N_NOISY), f32)], axis=0)
    padn = jnp.concatenate(
        [jnp.full((2, _N1_PAD - _N_NOISY), 1e6, f32),
         jnp.zeros((6, _N1_PAD - _N_NOISY), f32)], axis=0)
    xyc = jnp.concatenate([xyT, padn], axis=1)
    xyc = xyc.reshape(8, _N1_NCHUNK, _N1_CHUNK).transpose(1, 0, 2)

    cT = jnp.concatenate([pc.T, jnp.zeros((5, _N_CLEAN), f32)], axis=0)
    padc = jnp.concatenate(
        [jnp.full((2, _N2_PAD - _N_CLEAN), 1e6, f32),
         jnp.zeros((6, _N2_PAD - _N_CLEAN), f32)], axis=0)
    cc = jnp.concatenate([cT, padc], axis=1)
    cc = cc.reshape(8, _N2_NCHUNK, _N2_CHUNK).transpose(1, 0, 2)

    idx = _knn1(qxy8, xyc)
    idxf = idx.reshape(_NQ * _K)
    frames16 = _sc_gather(table128, idxf)[:, :16]
    fq = jnp.concatenate(
        [frames16[:, :2], jnp.zeros((_NQ * _K, 6), f32)], axis=1)
    mz = _knn2(fq, cc)

    w1p = jnp.pad(feat_W1, ((0, 13), (0, 0)))
    wina = jnp.pad(score_Win[:3], ((0, 13), (0, 0)))
    winb = score_Win[3:]
    wop = jnp.pad(score_Wout, ((0, 0), (0, 127)))
    loss = _mlp(q16, frames16, mz,
                w1p, feat_b1.reshape(1, _F), feat_W2, feat_b2.reshape(1, _F),
                wina, winb, score_bin.reshape(1, _HID),
                score_Wb, score_bb.reshape(_NBLK, 1, _HID),
                wop, score_bout.reshape(1, 1))
    return loss[0, 0]


# speedup vs baseline: 2.1536x; 1.2801x over previous
import jax
import jax.numpy as jnp
from jax import lax
from jax.experimental import pallas as pl
from jax.experimental.pallas import tpu as pltpu
from jax.experimental.pallas import tpu_sc as plsc

_N_NOISY = 50000
_N_CLEAN = 52000
_NQ = 128
_K = 32
_C = 4
_F = 128
_HID = 128
_NBLK = 4
_SIGMA = 0.01

_N1_PAD = 51200
_N1_CHUNK = 12800
_N1_NCHUNK = 4
_N2_PAD = 52224
_N2_CHUNK = 6528
_N2_NCHUNK = 8
_QTILE = 256
_BIGF = 3.0e38
_BIGI = 2 ** 30


def _knn1_body(q_ref, xyc_ref, idx_ref):
    q = q_ref[...]
    qsq = jnp.sum(q * q, axis=1, keepdims=True)
    nslot = _N1_NCHUNK * _K
    slot_iota = lax.broadcasted_iota(jnp.int32, (1, nslot), 1)

    def chunk_body(c, carry):
        cv, ci = carry
        ch = xyc_ref[c]
        psq = jnp.sum(ch * ch, axis=0, keepdims=True)
        d2 = qsq - 2.0 * jnp.dot(q, ch, preferred_element_type=jnp.float32) + psq
        iota = lax.broadcasted_iota(jnp.int32, d2.shape, 1)

        def k_body(k, kc):
            d2k, cv, ci = kc
            m = jnp.min(d2k, axis=1, keepdims=True)
            sel0 = d2k == m
            pos = jnp.min(jnp.where(sel0, iota, _BIGI), axis=1, keepdims=True)
            d2k = jnp.where(sel0, _BIGF, d2k)
            sel = slot_iota == (c * _K + k)
            cv = jnp.where(sel, m, cv)
            ci = jnp.where(sel, pos + c * _N1_CHUNK, ci)
            return d2k, cv, ci

        _, cv, ci = lax.fori_loop(0, _K, k_body, (d2, cv, ci))
        return cv, ci

    cv0 = jnp.full((_NQ, nslot), _BIGF, jnp.float32)
    ci0 = jnp.zeros((_NQ, nslot), jnp.int32)
    cv, ci = lax.fori_loop(0, _N1_NCHUNK, chunk_body, (cv0, ci0))

    iota2 = lax.broadcasted_iota(jnp.int32, cv.shape, 1)
    kiota = lax.broadcasted_iota(jnp.int32, (1, _K), 1)

    def out_body(k, oc):
        cv, out = oc
        m = jnp.min(cv, axis=1, keepdims=True)
        pos = jnp.min(jnp.where(cv == m, iota2, _BIGI), axis=1, keepdims=True)
        sel = iota2 == pos
        gi = jnp.sum(jnp.where(sel, ci, 0), axis=1, keepdims=True)
        out = jnp.where(kiota == k, gi, out)
        cv = jnp.where(sel, _BIGF, cv)
        return cv, out

    _, out = lax.fori_loop(0, _K, out_body,
                           (cv, jnp.zeros((_NQ, _K), jnp.int32)))
    idx_ref[...] = out


def _knn1(qxy8, xyc):
    return pl.pallas_call(
        _knn1_body,
        out_shape=jax.ShapeDtypeStruct((_NQ, _K), jnp.int32),
    )(qxy8, xyc)


def _sc_gather_body(table_hbm, idx_hbm, out_hbm, idx_v, rows_v, sem):
    wid = lax.axis_index("s") * 2 + lax.axis_index("c")
    base = wid * 128
    pltpu.sync_copy(idx_hbm.at[pl.ds(base, 128)], idx_v)
    pltpu.async_copy(table_hbm.at[idx_v], rows_v, sem).wait()
    pltpu.sync_copy(rows_v, out_hbm.at[pl.ds(base, 128)])


def _sc_gather(table128, idxf):
    mesh = plsc.VectorSubcoreMesh(core_axis_name="c", subcore_axis_name="s")
    k = pl.kernel(
        _sc_gather_body,
        out_type=jax.ShapeDtypeStruct((_NQ * _K, 128), jnp.float32),
        mesh=mesh,
        scratch_types=[
            pltpu.VMEM((128,), jnp.int32),
            pltpu.VMEM((128, 128), jnp.float32),
            pltpu.SemaphoreType.DMA,
        ],
    )
    return k(table128, idxf)


def _knn2_body(fq_ref, cc_ref, mz_ref):
    q = fq_ref[...]
    qsq = jnp.sum(q * q, axis=1, keepdims=True)
    nslot = _N2_NCHUNK * _C
    slot_iota = lax.broadcasted_iota(jnp.int32, (1, nslot), 1)

    def chunk_body(c, carry):
        cv, cz = carry
        ch = cc_ref[c]
        xy = ch[0:2]
        psq = jnp.sum(xy * xy, axis=0, keepdims=True)
        zrow = ch[2:3]
        d2 = qsq - 2.0 * jnp.dot(q, ch, preferred_element_type=jnp.float32) + psq

        def k_body(k, kc):
            d2k, cv, cz = kc
            m = jnp.min(d2k, axis=1, keepdims=True)
            sel0 = d2k == m
            z = jnp.sum(jnp.where(sel0, zrow, 0.0), axis=1, keepdims=True)
            d2k = jnp.where(sel0, _BIGF, d2k)
            sel = slot_iota == (c * _C + k)
            cv = jnp.where(sel, m, cv)
            cz = jnp.where(sel, z, cz)
            return d2k, cv, cz

        _, cv, cz = lax.fori_loop(0, _C, k_body, (d2, cv, cz))
        return cv, cz

    cv0 = jnp.full((_QTILE, nslot), _BIGF, jnp.float32)
    cz0 = jnp.zeros((_QTILE, nslot), jnp.float32)
    cv, cz = lax.fori_loop(0, _N2_NCHUNK, chunk_body, (cv0, cz0))

    iota2 = lax.broadcasted_iota(jnp.int32, cv.shape, 1)

    def out_body(k, oc):
        cv, acc = oc
        m = jnp.min(cv, axis=1, keepdims=True)
        pos = jnp.min(jnp.where(cv == m, iota2, _BIGI), axis=1, keepdims=True)
        sel = iota2 == pos
        acc = acc + jnp.sum(jnp.where(sel, cz, 0.0), axis=1, keepdims=True)
        cv = jnp.where(sel, _BIGF, cv)
        return cv, acc

    _, acc = lax.fori_loop(0, _C, out_body,
                           (cv, jnp.zeros((_QTILE, 1), jnp.float32)))
    mz_ref[...] = acc * (1.0 / _C)


def _knn2(fq, cc):
    n = _NQ * _K
    return pl.pallas_call(
        _knn2_body,
        grid=(n // _QTILE,),
        in_specs=[
            pl.BlockSpec((_QTILE, 8), lambda i: (i, 0)),
            pl.BlockSpec((_N2_NCHUNK, 8, _N2_CHUNK), lambda i: (0, 0, 0)),
        ],
        out_specs=pl.BlockSpec((_QTILE, 1), lambda i: (i, 0)),
        out_shape=jax.ShapeDtypeStruct((n, 1), jnp.float32),
    )(fq, cc)


def _mlp_body(q_ref, fr_ref, mz_ref, w1_ref, b1_ref, w2_ref, b2_ref,
              wa_ref, wbf_ref, bin_ref, wbk_ref, bbk_ref, wo_ref, bo_ref,
              out_ref):
    q = q_ref[...]
    h = jnp.maximum(jnp.dot(q, w1_ref[...], preferred_element_type=jnp.float32)
                    + b1_ref[...], 0.0)
    feat = jnp.dot(h, w2_ref[...], preferred_element_type=jnp.float32) + b2_ref[...]
    fr = fr_ref[...]

    def rep(a):
        return jnp.reshape(
            jnp.broadcast_to(a[:, None, :], (_NQ, _K, a.shape[1])),
            (_NQ * _K, a.shape[1]))

    cfeat = rep(feat)
    x = fr - rep(q)
    hs = jnp.maximum(
        jnp.dot(x, wa_ref[...], preferred_element_type=jnp.float32)
        + jnp.dot(cfeat, wbf_ref[...], preferred_element_type=jnp.float32)
        + bin_ref[...], 0.0)
    for i in range(_NBLK):
        hs = jnp.maximum(
            jnp.dot(hs, wbk_ref[i], preferred_element_type=jnp.float32)
            + bbk_ref[i], 0.0) + hs
    gp = jnp.dot(hs, wo_ref[...], preferred_element_type=jnp.float32)
    colmask = lax.broadcasted_iota(jnp.int32, gp.shape, 1) == 0
    z = fr[:, 2:3]
    gt = mz_ref[...] - z
    diff = jnp.where(colmask, gt - (gp + bo_ref[...]), 0.0)
    total = jnp.sum(diff * diff)
    out_ref[...] = jnp.reshape(0.5 * total * (1.0 / _SIGMA) / (_NQ * _K), (1, 1))


def _mlp(q16, frames16, mz, w1p, b1, w2, b2, wina, winb, binp, wbk, bbk, wop, bop):
    return pl.pallas_call(
        _mlp_body,
        out_shape=jax.ShapeDtypeStruct((1, 1), jnp.float32),
    )(q16, frames16, mz, w1p, b1, w2, b2, wina, winb, binp, wbk, bbk, wop, bop)


def kernel(pcl_noisy, pcl_clean, feat_W1, feat_b1, feat_W2, feat_b2,
           score_Win, score_bin, score_Wb, score_bb, score_Wout, score_bout):
    f32 = jnp.float32
    pn = pcl_noisy[0]
    pc = pcl_clean[0]

    table128 = jnp.pad(pn, ((0, 0), (0, 125)))
    q16 = jnp.pad(pn[:_NQ], ((0, 0), (0, 13)))

    def aug_queries(xy, n):
        return jnp.concatenate([xy, jnp.zeros((n, 6), f32)], axis=1)

    qxy8 = aug_queries(pn[:_NQ, :2], _NQ)

    def aug_points(rows, npts, npad, nchunk, chunk):
        nr = rows.shape[0]
        body = jnp.concatenate([rows, jnp.zeros((8 - nr, npts), f32)], axis=0)
        w = npad - npts
        pad = jnp.concatenate(
            [jnp.full((2, w), 1e6, f32), jnp.zeros((6, w), f32)], axis=0)
        out = jnp.concatenate([body, pad], axis=1)
        return out.reshape(8, nchunk, chunk).transpose(1, 0, 2)

    xyc = aug_points(pn[:, :2].T, _N_NOISY, _N1_PAD, _N1_NCHUNK, _N1_CHUNK)
    cc = aug_points(pc.T, _N_CLEAN, _N2_PAD, _N2_NCHUNK, _N2_CHUNK)

    idx = _knn1(qxy8, xyc)
    idxf = idx.reshape(_NQ * _K)
    frames16 = _sc_gather(table128, idxf)[:, :16]
    fq = aug_queries(frames16[:, :2], _NQ * _K)
    mz = _knn2(fq, cc)

    w1p = jnp.pad(feat_W1, ((0, 13), (0, 0)))
    wina = jnp.pad(score_Win[:3], ((0, 13), (0, 0)))
    winb = score_Win[3:]
    wop = jnp.pad(score_Wout, ((0, 0), (0, 127)))
    loss = _mlp(q16, frames16, mz,
                w1p, feat_b1.reshape(1, _F), feat_W2, feat_b2.reshape(1, _F),
                wina, winb, score_bin.reshape(1, _HID),
                score_Wb, score_bb.reshape(_NBLK, 1, _HID),
                wop, score_bout.reshape(1, 1))
    return loss[0, 0]


# speedup vs baseline: 2.1592x; 1.0026x over previous
import jax
import jax.numpy as jnp
from jax import lax
from jax.experimental import pallas as pl
from jax.experimental.pallas import tpu as pltpu
from jax.experimental.pallas import tpu_sc as plsc

_N_NOISY = 50000
_N_CLEAN = 52000
_NQ = 128
_K = 32
_C = 4
_F = 128
_HID = 128
_NBLK = 4
_SIGMA = 0.01

_N1_PAD = 51200
_N1_CHUNK = 25600
_N1_NCHUNK = 2
_N2_PAD = 52224
_N2_CHUNK = 6528
_N2_NCHUNK = 8
_QTILE = 512
_BIGF = 3.0e38
_BIGI = 2 ** 30


def _knn1_body(q_ref, xyc_ref, idx_ref):
    q = q_ref[...]
    qsq = jnp.sum(q * q, axis=1, keepdims=True)
    nslot = _N1_NCHUNK * _K
    slot_iota = lax.broadcasted_iota(jnp.int32, (1, nslot), 1)

    def chunk_body(c, carry):
        cv, ci = carry
        ch = xyc_ref[c]
        psq = jnp.sum(ch * ch, axis=0, keepdims=True)
        d2 = qsq - 2.0 * jnp.dot(q, ch, preferred_element_type=jnp.float32) + psq
        iota = lax.broadcasted_iota(jnp.int32, d2.shape, 1)

        def k_body(k, kc):
            d2k, cv, ci = kc
            m = jnp.min(d2k, axis=1, keepdims=True)
            sel0 = d2k == m
            pos = jnp.min(jnp.where(sel0, iota, _BIGI), axis=1, keepdims=True)
            d2k = jnp.where(sel0, _BIGF, d2k)
            sel = slot_iota == (c * _K + k)
            cv = jnp.where(sel, m, cv)
            ci = jnp.where(sel, pos + c * _N1_CHUNK, ci)
            return d2k, cv, ci

        _, cv, ci = lax.fori_loop(0, _K, k_body, (d2, cv, ci))
        return cv, ci

    cv0 = jnp.full((_NQ, nslot), _BIGF, jnp.float32)
    ci0 = jnp.zeros((_NQ, nslot), jnp.int32)
    cv, ci = lax.fori_loop(0, _N1_NCHUNK, chunk_body, (cv0, ci0))

    iota2 = lax.broadcasted_iota(jnp.int32, cv.shape, 1)
    kiota = lax.broadcasted_iota(jnp.int32, (1, _K), 1)

    def out_body(k, oc):
        cv, out = oc
        m = jnp.min(cv, axis=1, keepdims=True)
        pos = jnp.min(jnp.where(cv == m, iota2, _BIGI), axis=1, keepdims=True)
        sel = iota2 == pos
        gi = jnp.sum(jnp.where(sel, ci, 0), axis=1, keepdims=True)
        out = jnp.where(kiota == k, gi, out)
        cv = jnp.where(sel, _BIGF, cv)
        return cv, out

    _, out = lax.fori_loop(0, _K, out_body,
                           (cv, jnp.zeros((_NQ, _K), jnp.int32)))
    idx_ref[...] = out


def _knn1(qxy8, xyc):
    return pl.pallas_call(
        _knn1_body,
        out_shape=jax.ShapeDtypeStruct((_NQ, _K), jnp.int32),
    )(qxy8, xyc)


def _sc_gather_body(table_hbm, idx_hbm, out_hbm, idx_v, rows_v, sem):
    wid = lax.axis_index("s") * 2 + lax.axis_index("c")
    base = wid * 128
    pltpu.sync_copy(idx_hbm.at[pl.ds(base, 128)], idx_v)
    pltpu.async_copy(table_hbm.at[idx_v], rows_v, sem).wait()
    pltpu.sync_copy(rows_v, out_hbm.at[pl.ds(base, 128)])


def _sc_gather(table128, idxf):
    mesh = plsc.VectorSubcoreMesh(core_axis_name="c", subcore_axis_name="s")
    k = pl.kernel(
        _sc_gather_body,
        out_type=jax.ShapeDtypeStruct((_NQ * _K, 128), jnp.float32),
        mesh=mesh,
        scratch_types=[
            pltpu.VMEM((128,), jnp.int32),
            pltpu.VMEM((128, 128), jnp.float32),
            pltpu.SemaphoreType.DMA,
        ],
    )
    return k(table128, idxf)


def _knn2_body(fq_ref, cc_ref, mz_ref):
    q = fq_ref[...]
    qsq = jnp.sum(q * q, axis=1, keepdims=True)
    nslot = _N2_NCHUNK * _C
    slot_iota = lax.broadcasted_iota(jnp.int32, (1, nslot), 1)

    def chunk_body(c, carry):
        cv, cz = carry
        ch = cc_ref[c]
        xy = ch[0:2]
        psq = jnp.sum(xy * xy, axis=0, keepdims=True)
        zrow = ch[2:3]
        d2 = qsq - 2.0 * jnp.dot(q, ch, preferred_element_type=jnp.float32) + psq

        def k_body(k, kc):
            d2k, cv, cz = kc
            m = jnp.min(d2k, axis=1, keepdims=True)
            sel0 = d2k == m
            z = jnp.sum(jnp.where(sel0, zrow, 0.0), axis=1, keepdims=True)
            d2k = jnp.where(sel0, _BIGF, d2k)
            sel = slot_iota == (c * _C + k)
            cv = jnp.where(sel, m, cv)
            cz = jnp.where(sel, z, cz)
            return d2k, cv, cz

        _, cv, cz = lax.fori_loop(0, _C, k_body, (d2, cv, cz))
        return cv, cz

    cv0 = jnp.full((_QTILE, nslot), _BIGF, jnp.float32)
    cz0 = jnp.zeros((_QTILE, nslot), jnp.float32)
    cv, cz = lax.fori_loop(0, _N2_NCHUNK, chunk_body, (cv0, cz0))

    iota2 = lax.broadcasted_iota(jnp.int32, cv.shape, 1)

    def out_body(k, oc):
        cv, acc = oc
        m = jnp.min(cv, axis=1, keepdims=True)
        pos = jnp.min(jnp.where(cv == m, iota2, _BIGI), axis=1, keepdims=True)
        sel = iota2 == pos
        acc = acc + jnp.sum(jnp.where(sel, cz, 0.0), axis=1, keepdims=True)
        cv = jnp.where(sel, _BIGF, cv)
        return cv, acc

    _, acc = lax.fori_loop(0, _C, out_body,
                           (cv, jnp.zeros((_QTILE, 1), jnp.float32)))
    mz_ref[...] = acc * (1.0 / _C)


def _knn2(fq, cc):
    n = _NQ * _K
    return pl.pallas_call(
        _knn2_body,
        grid=(n // _QTILE,),
        in_specs=[
            pl.BlockSpec((_QTILE, 8), lambda i: (i, 0)),
            pl.BlockSpec((_N2_NCHUNK, 8, _N2_CHUNK), lambda i: (0, 0, 0)),
        ],
        out_specs=pl.BlockSpec((_QTILE, 1), lambda i: (i, 0)),
        out_shape=jax.ShapeDtypeStruct((n, 1), jnp.float32),
    )(fq, cc)


def _mlp_body(q_ref, fr_ref, mz_ref, w1_ref, b1_ref, w2_ref, b2_ref,
              wa_ref, wbf_ref, bin_ref, wbk_ref, bbk_ref, wo_ref, bo_ref,
              out_ref):
    q = q_ref[...]
    h = jnp.maximum(jnp.dot(q, w1_ref[...], preferred_element_type=jnp.float32)
                    + b1_ref[...], 0.0)
    feat = jnp.dot(h, w2_ref[...], preferred_element_type=jnp.float32) + b2_ref[...]
    fr = fr_ref[...]

    def rep(a):
        return jnp.reshape(
            jnp.broadcast_to(a[:, None, :], (_NQ, _K, a.shape[1])),
            (_NQ * _K, a.shape[1]))

    cfeat = rep(feat)
    x = fr - rep(q)
    hs = jnp.maximum(
        jnp.dot(x, wa_ref[...], preferred_element_type=jnp.float32)
        + jnp.dot(cfeat, wbf_ref[...], preferred_element_type=jnp.float32)
        + bin_ref[...], 0.0)
    for i in range(_NBLK):
        hs = jnp.maximum(
            jnp.dot(hs, wbk_ref[i], preferred_element_type=jnp.float32)
            + bbk_ref[i], 0.0) + hs
    gp = jnp.dot(hs, wo_ref[...], preferred_element_type=jnp.float32)
    colmask = lax.broadcasted_iota(jnp.int32, gp.shape, 1) == 0
    z = fr[:, 2:3]
    gt = mz_ref[...] - z
    diff = jnp.where(colmask, gt - (gp + bo_ref[...]), 0.0)
    total = jnp.sum(diff * diff)
    out_ref[...] = jnp.reshape(0.5 * total * (1.0 / _SIGMA) / (_NQ * _K), (1, 1))


def _mlp(q16, frames16, mz, w1p, b1, w2, b2, wina, winb, binp, wbk, bbk, wop, bop):
    return pl.pallas_call(
        _mlp_body,
        out_shape=jax.ShapeDtypeStruct((1, 1), jnp.float32),
    )(q16, frames16, mz, w1p, b1, w2, b2, wina, winb, binp, wbk, bbk, wop, bop)


def kernel(pcl_noisy, pcl_clean, feat_W1, feat_b1, feat_W2, feat_b2,
           score_Win, score_bin, score_Wb, score_bb, score_Wout, score_bout):
    f32 = jnp.float32
    pn = pcl_noisy[0]
    pc = pcl_clean[0]

    table128 = jnp.pad(pn, ((0, 0), (0, 125)))
    q16 = jnp.pad(pn[:_NQ], ((0, 0), (0, 13)))

    def aug_queries(xy, n):
        return jnp.concatenate([xy, jnp.zeros((n, 6), f32)], axis=1)

    qxy8 = aug_queries(pn[:_NQ, :2], _NQ)

    def aug_points(rows, npts, npad, nchunk, chunk):
        nr = rows.shape[0]
        body = jnp.concatenate([rows, jnp.zeros((8 - nr, npts), f32)], axis=0)
        w = npad - npts
        pad = jnp.concatenate(
            [jnp.full((2, w), 1e6, f32), jnp.zeros((6, w), f32)], axis=0)
        out = jnp.concatenate([body, pad], axis=1)
        return out.reshape(8, nchunk, chunk).transpose(1, 0, 2)

    xyc = aug_points(pn[:, :2].T, _N_NOISY, _N1_PAD, _N1_NCHUNK, _N1_CHUNK)
    cc = aug_points(pc.T, _N_CLEAN, _N2_PAD, _N2_NCHUNK, _N2_CHUNK)

    idx = _knn1(qxy8, xyc)
    idxf = idx.reshape(_NQ * _K)
    frames16 = _sc_gather(table128, idxf)[:, :16]
    fq = aug_queries(frames16[:, :2], _NQ * _K)
    mz = _knn2(fq, cc)

    w1p = jnp.pad(feat_W1, ((0, 13), (0, 0)))
    wina = jnp.pad(score_Win[:3], ((0, 13), (0, 0)))
    winb = score_Win[3:]
    wop = jnp.pad(score_Wout, ((0, 0), (0, 127)))
    loss = _mlp(q16, frames16, mz,
                w1p, feat_b1.reshape(1, _F), feat_W2, feat_b2.reshape(1, _F),
                wina, winb, score_bin.reshape(1, _HID),
                score_Wb, score_bb.reshape(_NBLK, 1, _HID),
                wop, score_bout.reshape(1, 1))
    return loss[0, 0]


# speedup vs baseline: 3.8532x; 1.7845x over previous
import jax
import jax.numpy as jnp
from jax import lax
from jax.experimental import pallas as pl
from jax.experimental.pallas import tpu as pltpu
from jax.experimental.pallas import tpu_sc as plsc

_N_NOISY = 50000
_N_CLEAN = 52000
_NQ = 128
_K = 32
_C = 4
_F = 128
_HID = 128
_NBLK = 4
_SIGMA = 0.01

_N1_PAD = 51200
_N1_CHUNK = 25600
_N1_NCHUNK = 2
_N2_PAD = 52224
_N2_CHUNK = 6528
_N2_NCHUNK = 8
_QTILE = 512
_BIGF = 3.0e38
_BIGI = 2 ** 30


def _knn1_body(q_ref, xyc_ref, idx_ref):
    q = q_ref[...]
    qsq = jnp.sum(q * q, axis=1, keepdims=True)
    nslot = _N1_NCHUNK * _K
    slot_iota = lax.broadcasted_iota(jnp.int32, (1, nslot), 1)

    def chunk_body(c, carry):
        cv, ci = carry
        ch = xyc_ref[c]
        psq = jnp.sum(ch * ch, axis=0, keepdims=True)
        d2 = qsq - 2.0 * jnp.dot(q, ch, preferred_element_type=jnp.float32) + psq
        iota = lax.broadcasted_iota(jnp.int32, d2.shape, 1)

        def k_body(k, kc):
            d2k, cv, ci = kc
            m = jnp.min(d2k, axis=1, keepdims=True)
            sel0 = d2k == m
            pos = jnp.min(jnp.where(sel0, iota, _BIGI), axis=1, keepdims=True)
            d2k = jnp.where(sel0, _BIGF, d2k)
            sel = slot_iota == (c * _K + k)
            cv = jnp.where(sel, m, cv)
            ci = jnp.where(sel, pos + c * _N1_CHUNK, ci)
            return d2k, cv, ci

        _, cv, ci = lax.fori_loop(0, _K, k_body, (d2, cv, ci))
        return cv, ci

    cv0 = jnp.full((_NQ, nslot), _BIGF, jnp.float32)
    ci0 = jnp.zeros((_NQ, nslot), jnp.int32)
    cv, ci = lax.fori_loop(0, _N1_NCHUNK, chunk_body, (cv0, ci0))

    iota2 = lax.broadcasted_iota(jnp.int32, cv.shape, 1)
    kiota = lax.broadcasted_iota(jnp.int32, (1, _K), 1)

    def out_body(k, oc):
        cv, out = oc
        m = jnp.min(cv, axis=1, keepdims=True)
        pos = jnp.min(jnp.where(cv == m, iota2, _BIGI), axis=1, keepdims=True)
        sel = iota2 == pos
        gi = jnp.sum(jnp.where(sel, ci, 0), axis=1, keepdims=True)
        out = jnp.where(kiota == k, gi, out)
        cv = jnp.where(sel, _BIGF, cv)
        return cv, out

    _, out = lax.fori_loop(0, _K, out_body,
                           (cv, jnp.zeros((_NQ, _K), jnp.int32)))
    idx_ref[...] = out


def _knn1(qxy8, xyc):
    return pl.pallas_call(
        _knn1_body,
        out_shape=jax.ShapeDtypeStruct((_NQ, _K), jnp.int32),
    )(qxy8, xyc)


def _sc_gather_body(table_hbm, idx_hbm, out_hbm, idx_v, rows_v, sem):
    wid = lax.axis_index("s") * 2 + lax.axis_index("c")
    base = wid * 128
    pltpu.sync_copy(idx_hbm.at[pl.ds(base, 128)], idx_v)
    pltpu.async_copy(table_hbm.at[idx_v], rows_v, sem).wait()
    pltpu.sync_copy(rows_v, out_hbm.at[pl.ds(base, 128)])


def _sc_gather(table128, idxf):
    mesh = plsc.VectorSubcoreMesh(core_axis_name="c", subcore_axis_name="s")
    k = pl.kernel(
        _sc_gather_body,
        out_type=jax.ShapeDtypeStruct((_NQ * _K, 128), jnp.float32),
        mesh=mesh,
        scratch_types=[
            pltpu.VMEM((128,), jnp.int32),
            pltpu.VMEM((128, 128), jnp.float32),
            pltpu.SemaphoreType.DMA,
        ],
    )
    return k(table128, idxf)


def _knn2_body(fq_ref, cc_ref, mz_ref):
    q = fq_ref[...]
    qsq = jnp.sum(q * q, axis=1, keepdims=True)
    nb = _N2_CHUNK // 128

    def chunk_body(c, carry):
        r1, r2, r3, r4, z1, z2, z3, z4 = carry
        ch = cc_ref[c]
        xy = ch[0:2]
        psq = jnp.sum(xy * xy, axis=0, keepdims=True)
        zrow = ch[2:3]
        d2 = qsq - 2.0 * jnp.dot(q, ch, preferred_element_type=jnp.float32) + psq
        r = [r1, r2, r3, r4]
        z = [z1, z2, z3, z4]
        for b in range(nb):
            e = d2[:, b * 128:(b + 1) * 128]
            ez = zrow[:, b * 128:(b + 1) * 128]
            for i in range(4):
                cond = e < r[i]
                nri = jnp.where(cond, e, r[i])
                nzi = jnp.where(cond, ez, z[i])
                e = jnp.where(cond, r[i], e)
                ez = jnp.where(cond, z[i], ez)
                r[i], z[i] = nri, nzi
        return tuple(r) + tuple(z)

    full = lambda v: jnp.full((_QTILE, 128), v, jnp.float32)
    init = (full(_BIGF),) * 4 + (full(0.0),) * 4
    res = lax.fori_loop(0, _N2_NCHUNK, chunk_body, init)
    cv = jnp.concatenate(res[:4], axis=1)
    cz = jnp.concatenate(res[4:], axis=1)

    iota2 = lax.broadcasted_iota(jnp.int32, cv.shape, 1)

    def out_body(k, oc):
        cv, acc = oc
        m = jnp.min(cv, axis=1, keepdims=True)
        pos = jnp.min(jnp.where(cv == m, iota2, _BIGI), axis=1, keepdims=True)
        sel = iota2 == pos
        acc = acc + jnp.sum(jnp.where(sel, cz, 0.0), axis=1, keepdims=True)
        cv = jnp.where(sel, _BIGF, cv)
        return cv, acc

    _, acc = lax.fori_loop(0, _C, out_body,
                           (cv, jnp.zeros((_QTILE, 1), jnp.float32)))
    mz_ref[...] = acc * (1.0 / _C)


def _knn2(fq, cc):
    n = _NQ * _K
    return pl.pallas_call(
        _knn2_body,
        grid=(n // _QTILE,),
        in_specs=[
            pl.BlockSpec((_QTILE, 8), lambda i: (i, 0)),
            pl.BlockSpec((_N2_NCHUNK, 8, _N2_CHUNK), lambda i: (0, 0, 0)),
        ],
        out_specs=pl.BlockSpec((_QTILE, 1), lambda i: (i, 0)),
        out_shape=jax.ShapeDtypeStruct((n, 1), jnp.float32),
    )(fq, cc)


def _mlp_body(q_ref, fr_ref, mz_ref, w1_ref, b1_ref, w2_ref, b2_ref,
              wa_ref, wbf_ref, bin_ref, wbk_ref, bbk_ref, wo_ref, bo_ref,
              out_ref):
    q = q_ref[...]
    h = jnp.maximum(jnp.dot(q, w1_ref[...], preferred_element_type=jnp.float32)
                    + b1_ref[...], 0.0)
    feat = jnp.dot(h, w2_ref[...], preferred_element_type=jnp.float32) + b2_ref[...]
    fr = fr_ref[...]

    def rep(a):
        return jnp.reshape(
            jnp.broadcast_to(a[:, None, :], (_NQ, _K, a.shape[1])),
            (_NQ * _K, a.shape[1]))

    cfeat = rep(feat)
    x = fr - rep(q)
    hs = jnp.maximum(
        jnp.dot(x, wa_ref[...], preferred_element_type=jnp.float32)
        + jnp.dot(cfeat, wbf_ref[...], preferred_element_type=jnp.float32)
        + bin_ref[...], 0.0)
    for i in range(_NBLK):
        hs = jnp.maximum(
            jnp.dot(hs, wbk_ref[i], preferred_element_type=jnp.float32)
            + bbk_ref[i], 0.0) + hs
    gp = jnp.dot(hs, wo_ref[...], preferred_element_type=jnp.float32)
    colmask = lax.broadcasted_iota(jnp.int32, gp.shape, 1) == 0
    z = fr[:, 2:3]
    gt = mz_ref[...] - z
    diff = jnp.where(colmask, gt - (gp + bo_ref[...]), 0.0)
    total = jnp.sum(diff * diff)
    out_ref[...] = jnp.reshape(0.5 * total * (1.0 / _SIGMA) / (_NQ * _K), (1, 1))


def _mlp(q16, frames16, mz, w1p, b1, w2, b2, wina, winb, binp, wbk, bbk, wop, bop):
    return pl.pallas_call(
        _mlp_body,
        out_shape=jax.ShapeDtypeStruct((1, 1), jnp.float32),
    )(q16, frames16, mz, w1p, b1, w2, b2, wina, winb, binp, wbk, bbk, wop, bop)


def kernel(pcl_noisy, pcl_clean, feat_W1, feat_b1, feat_W2, feat_b2,
           score_Win, score_bin, score_Wb, score_bb, score_Wout, score_bout):
    f32 = jnp.float32
    pn = pcl_noisy[0]
    pc = pcl_clean[0]

    table128 = jnp.pad(pn, ((0, 0), (0, 125)))
    q16 = jnp.pad(pn[:_NQ], ((0, 0), (0, 13)))

    def aug_queries(xy, n):
        return jnp.concatenate([xy, jnp.zeros((n, 6), f32)], axis=1)

    qxy8 = aug_queries(pn[:_NQ, :2], _NQ)

    def aug_points(rows, npts, npad, nchunk, chunk):
        nr = rows.shape[0]
        body = jnp.concatenate([rows, jnp.zeros((8 - nr, npts), f32)], axis=0)
        w = npad - npts
        pad = jnp.concatenate(
            [jnp.full((2, w), 1e6, f32), jnp.zeros((6, w), f32)], axis=0)
        out = jnp.concatenate([body, pad], axis=1)
        return out.reshape(8, nchunk, chunk).transpose(1, 0, 2)

    xyc = aug_points(pn[:, :2].T, _N_NOISY, _N1_PAD, _N1_NCHUNK, _N1_CHUNK)
    cc = aug_points(pc.T, _N_CLEAN, _N2_PAD, _N2_NCHUNK, _N2_CHUNK)

    idx = _knn1(qxy8, xyc)
    idxf = idx.reshape(_NQ * _K)
    frames16 = _sc_gather(table128, idxf)[:, :16]
    fq = aug_queries(frames16[:, :2], _NQ * _K)
    mz = _knn2(fq, cc)

    w1p = jnp.pad(feat_W1, ((0, 13), (0, 0)))
    wina = jnp.pad(score_Win[:3], ((0, 13), (0, 0)))
    winb = score_Win[3:]
    wop = jnp.pad(score_Wout, ((0, 0), (0, 127)))
    loss = _mlp(q16, frames16, mz,
                w1p, feat_b1.reshape(1, _F), feat_W2, feat_b2.reshape(1, _F),
                wina, winb, score_bin.reshape(1, _HID),
                score_Wb, score_bb.reshape(_NBLK, 1, _HID),
                wop, score_bout.reshape(1, 1))
    return loss[0, 0]
